# baseline (speedup 1.0000x reference)
import functools

import jax
import jax.numpy as jnp
from jax import lax
from jax.experimental import pallas as pl
from jax.experimental.pallas import tpu as pltpu
from jax.experimental.pallas import tpu_sc as plsc

D_MODEL = 1024
D_FF = 2048
NUM_EXPERTS = 8
TOP_K = 2
T = 2048
BT = 512
MAXT = T * TOP_K // BT + NUM_EXPERTS
PADTOT = MAXT * BT
NC = 2
NS = 16
NW = NC * NS
TPW = T // NW
HALF = TPW // 2


def _router_body(x_ref, wr_ref, pos0_ref, pos1_ref, w0_ref, w1_ref,
                 meta_ref, nact_ref, loss_ref):
    x = x_ref[...]
    logits = jnp.dot(x, wr_ref[...], preferred_element_type=jnp.float32)
    m = jnp.max(logits, axis=1, keepdims=True)
    ex = jnp.exp(logits - m)
    probs = ex / jnp.sum(ex, axis=1, keepdims=True)

    eidx = jax.lax.broadcasted_iota(jnp.int32, (T, NUM_EXPERTS), 1).astype(jnp.float32)
    v1 = jnp.max(probs, axis=1, keepdims=True)
    i1 = jnp.min(jnp.where(probs == v1, eidx, float(NUM_EXPERTS)), axis=1, keepdims=True)
    oh1 = (eidx == i1).astype(jnp.float32)
    probs_m = jnp.where(oh1 > 0, -jnp.inf, probs)
    v2 = jnp.max(probs_m, axis=1, keepdims=True)
    i2 = jnp.min(jnp.where(probs_m == v2, eidx, float(NUM_EXPERTS)), axis=1, keepdims=True)
    oh2 = (eidx == i2).astype(jnp.float32)

    s = v1 + v2
    w0_ref[...] = v1 / s
    w1_ref[...] = v2 / s

    me = jnp.sum(probs, axis=0) / float(T)
    disp = oh1 + oh2
    counts = jnp.sum(disp, axis=0, keepdims=True)
    loss_ref[...] = jnp.reshape(
        float(NUM_EXPERTS) * jnp.sum(me * (counts[0] / float(T * TOP_K))), (1, 1))

    ti = jax.lax.broadcasted_iota(jnp.int32, (T, T), 0)
    tj = jax.lax.broadcasted_iota(jnp.int32, (T, T), 1)
    tril = (tj < ti).astype(jnp.float32)
    R = jnp.dot(tril, disp, preferred_element_type=jnp.float32)

    n_tiles = jnp.ceil(counts * (1.0 / BT))
    a8 = jax.lax.broadcasted_iota(jnp.int32, (NUM_EXPERTS, NUM_EXPERTS), 0)
    b8 = jax.lax.broadcasted_iota(jnp.int32, (NUM_EXPERTS, NUM_EXPERTS), 1)
    incl8 = (a8 <= b8).astype(jnp.float32)
    ends = jnp.dot(n_tiles, incl8, preferred_element_type=jnp.float32)
    row_start = (ends - n_tiles) * float(BT)

    rank1 = jnp.sum(oh1 * R, axis=1, keepdims=True)
    rank2 = jnp.sum(oh2 * R, axis=1, keepdims=True)
    base1 = jnp.sum(oh1 * row_start, axis=1, keepdims=True)
    base2 = jnp.sum(oh2 * row_start, axis=1, keepdims=True)
    pos0_ref[...] = (base1 + rank1).astype(jnp.int32)
    pos1_ref[...] = (base2 + rank2).astype(jnp.int32)

    nact_ref[...] = ends[:, NUM_EXPERTS - 1:].astype(jnp.int32)

    e8 = jax.lax.broadcasted_iota(jnp.int32, (1, NUM_EXPERTS), 1).astype(jnp.float32)
    last_e = jnp.max(jnp.where(n_tiles > 0, e8, -1.0), axis=1, keepdims=True)
    ii = jax.lax.broadcasted_iota(jnp.int32, (32, NUM_EXPERTS), 0).astype(jnp.float32)
    raw = jnp.sum((ends <= ii).astype(jnp.float32), axis=1, keepdims=True)
    meta_ref[...] = jnp.minimum(raw, last_e).astype(jnp.int32)


def _router(x, Wr):
    return pl.pallas_call(
        _router_body,
        out_shape=(
            jax.ShapeDtypeStruct((T, 1), jnp.int32),
            jax.ShapeDtypeStruct((T, 1), jnp.int32),
            jax.ShapeDtypeStruct((T, 1), jnp.float32),
            jax.ShapeDtypeStruct((T, 1), jnp.float32),
            jax.ShapeDtypeStruct((32, 1), jnp.int32),
            jax.ShapeDtypeStruct((1, 1), jnp.int32),
            jax.ShapeDtypeStruct((1, 1), jnp.float32),
        ),
    )(x, Wr)


@functools.cache
def _sc_mesh():
    return plsc.VectorSubcoreMesh(core_axis_name="c", subcore_axis_name="s")


def _dispatch_body(x_hbm, posr_hbm, xg_hbm, idx_v, rows_a, rows_b,
                   sin_a, sin_b, sout):
    wid = lax.axis_index("s") * NC + lax.axis_index("c")
    pltpu.sync_copy(posr_hbm.at[wid], idx_v)
    base = wid * TPW
    in_a = pltpu.async_copy(x_hbm.at[pl.ds(base, HALF)], rows_a, sin_a)
    in_b = pltpu.async_copy(x_hbm.at[pl.ds(base + HALF, HALF)], rows_b, sin_b)
    in_a.wait()
    s0 = pltpu.async_copy(rows_a, xg_hbm.at[idx_v.at[0]], sout)
    s1 = pltpu.async_copy(rows_a, xg_hbm.at[idx_v.at[1]], sout)
    in_b.wait()
    s2 = pltpu.async_copy(rows_b, xg_hbm.at[idx_v.at[2]], sout)
    s3 = pltpu.async_copy(rows_b, xg_hbm.at[idx_v.at[3]], sout)
    s0.wait()
    s1.wait()
    s2.wait()
    s3.wait()


def _dispatch(x, posr):
    return pl.kernel(
        _dispatch_body,
        out_type=jax.ShapeDtypeStruct((PADTOT, D_MODEL), jnp.float32),
        mesh=_sc_mesh(),
        scratch_types=[
            pltpu.VMEM((4, HALF), jnp.int32),
            pltpu.VMEM((HALF, D_MODEL), jnp.float32),
            pltpu.VMEM((HALF, D_MODEL), jnp.float32),
            pltpu.SemaphoreType.DMA,
            pltpu.SemaphoreType.DMA,
            pltpu.SemaphoreType.DMA,
        ],
    )(x, posr)


def _gmm_body(meta_ref, nact_ref, xg_ref, w1_ref, b1_ref, w2_ref, b2_ref, y_ref):
    i = pl.program_id(0)

    @pl.when(i < nact_ref[0])
    def _():
        h = jnp.maximum(
            jnp.dot(xg_ref[...], w1_ref[0], preferred_element_type=jnp.float32)
            + b1_ref[0],
            0.0,
        )
        y_ref[...] = jnp.dot(h, w2_ref[0], preferred_element_type=jnp.float32) + b2_ref[0]


def _gmm(xg, W1, b1, W2, b2, meta, nact):
    grid_spec = pltpu.PrefetchScalarGridSpec(
        num_scalar_prefetch=2,
        grid=(MAXT,),
        in_specs=[
            pl.BlockSpec((BT, D_MODEL),
                         lambda i, meta, nact: (jnp.minimum(i, nact[0] - 1), 0)),
            pl.BlockSpec((1, D_MODEL, D_FF), lambda i, meta, nact: (meta[i], 0, 0)),
            pl.BlockSpec((1, 1, D_FF), lambda i, meta, nact: (meta[i], 0, 0)),
            pl.BlockSpec((1, D_FF, D_MODEL), lambda i, meta, nact: (meta[i], 0, 0)),
            pl.BlockSpec((1, 1, D_MODEL), lambda i, meta, nact: (meta[i], 0, 0)),
        ],
        out_specs=pl.BlockSpec((BT, D_MODEL), lambda i, meta, nact: (i, 0)),
    )
    return pl.pallas_call(
        _gmm_body,
        grid_spec=grid_spec,
        out_shape=jax.ShapeDtypeStruct((PADTOT, D_MODEL), jnp.float32),
    )(meta, nact, xg, W1, b1.reshape(NUM_EXPERTS, 1, D_FF), W2,
      b2.reshape(NUM_EXPERTS, 1, D_MODEL))


def _gather_body(y_hbm, x_hbm, posr_hbm, w_hbm, out_hbm,
                 idx_v, wv, xo_v, ya_v, yb_v, sin_x, sin_a, sin_b):
    wid = lax.axis_index("s") * NC + lax.axis_index("c")
    pltpu.sync_copy(posr_hbm.at[wid], idx_v)
    pltpu.sync_copy(w_hbm.at[wid], wv)
    base = wid * TPW
    for half in range(2):
        cx = pltpu.async_copy(x_hbm.at[pl.ds(base + half * HALF, HALF)], xo_v, sin_x)
        ca = pltpu.async_copy(y_hbm.at[idx_v.at[2 * half]], ya_v, sin_a)
        cb = pltpu.async_copy(y_hbm.at[idx_v.at[2 * half + 1]], yb_v, sin_b)
        cx.wait()
        ca.wait()
        cb.wait()

        def token_body(j, _):
            def vec_body(v, _):
                sl = pl.ds(v * 16, 16)
                wsl = pl.ds((half * HALF + j) * 16, 16)
                xo_v[j, sl] = (xo_v[j, sl] + wv[0, wsl] * ya_v[j, sl]
                               + wv[1, wsl] * yb_v[j, sl])
                return 0

            lax.fori_loop(0, D_MODEL // 16, vec_body, 0)
            return 0

        lax.fori_loop(0, HALF, token_body, 0)
        pltpu.sync_copy(xo_v, out_hbm.at[pl.ds(base + half * HALF, HALF)])


def _gather(y, x, posr, warr):
    return pl.kernel(
        _gather_body,
        out_type=jax.ShapeDtypeStruct((T, D_MODEL), jnp.float32),
        mesh=_sc_mesh(),
        scratch_types=[
            pltpu.VMEM((4, HALF), jnp.int32),
            pltpu.VMEM((2, TPW * 16), jnp.float32),
            pltpu.VMEM((HALF, D_MODEL), jnp.float32),
            pltpu.VMEM((HALF, D_MODEL), jnp.float32),
            pltpu.VMEM((HALF, D_MODEL), jnp.float32),
            pltpu.SemaphoreType.DMA,
            pltpu.SemaphoreType.DMA,
            pltpu.SemaphoreType.DMA,
        ],
    )(y, x, posr, warr)


def kernel(input_batch, Wr, W1, b1, W2, b2):
    B, S, D = input_batch.shape
    x = input_batch.reshape(T, D)

    pos0, pos1, w0, w1, meta, nact, loss = _router(x, Wr)

    p = jnp.concatenate(
        [pos0.reshape(NW, 2, HALF, 1), pos1.reshape(NW, 2, HALF, 1)], axis=3)
    posr = p.transpose(0, 1, 3, 2).reshape(NW, 4, HALF)

    warr = jnp.concatenate(
        [w0.reshape(NW, 1, TPW, 1), w1.reshape(NW, 1, TPW, 1)], axis=1)
    warr = jnp.broadcast_to(warr, (NW, 2, TPW, 16)).reshape(NW, 2, TPW * 16)

    xg = _dispatch(x, posr)
    y = _gmm(xg, W1, b1, W2, b2, meta.reshape(32), nact.reshape(1))
    out = _gather(y, x, posr, warr)

    return out.reshape(B, S, D), loss[0, 0]


# speedup vs baseline: 1.0983x; 1.0983x over previous
import functools

import jax
import jax.numpy as jnp
from jax import lax
from jax.experimental import pallas as pl
from jax.experimental.pallas import tpu as pltpu
from jax.experimental.pallas import tpu_sc as plsc

D_MODEL = 1024
D_FF = 2048
NUM_EXPERTS = 8
TOP_K = 2
T = 2048
BT = 512
MAXT = T * TOP_K // BT + NUM_EXPERTS
PADTOT = MAXT * BT
NC = 2
NS = 16
NW = NC * NS
TPW = T // NW
HALF = TPW // 2


def _router_body(x_ref, wr_ref, pos0_ref, pos1_ref, w0_ref, w1_ref,
                 meta_ref, nact_ref, loss_ref):
    x = x_ref[...]
    logits = jnp.dot(x, wr_ref[...], preferred_element_type=jnp.float32)
    m = jnp.max(logits, axis=1, keepdims=True)
    ex = jnp.exp(logits - m)
    probs = ex / jnp.sum(ex, axis=1, keepdims=True)

    eidx = jax.lax.broadcasted_iota(jnp.int32, (T, NUM_EXPERTS), 1).astype(jnp.float32)
    v1 = jnp.max(probs, axis=1, keepdims=True)
    i1 = jnp.min(jnp.where(probs == v1, eidx, float(NUM_EXPERTS)), axis=1, keepdims=True)
    oh1 = (eidx == i1).astype(jnp.float32)
    probs_m = jnp.where(oh1 > 0, -jnp.inf, probs)
    v2 = jnp.max(probs_m, axis=1, keepdims=True)
    i2 = jnp.min(jnp.where(probs_m == v2, eidx, float(NUM_EXPERTS)), axis=1, keepdims=True)
    oh2 = (eidx == i2).astype(jnp.float32)

    s = v1 + v2
    w0_ref[...] = v1 / s
    w1_ref[...] = v2 / s

    me = jnp.sum(probs, axis=0) / float(T)
    disp = oh1 + oh2
    counts = jnp.sum(disp, axis=0, keepdims=True)
    loss_ref[...] = jnp.reshape(
        float(NUM_EXPERTS) * jnp.sum(me * (counts[0] / float(T * TOP_K))), (1, 1))

    ti = jax.lax.broadcasted_iota(jnp.int32, (T, T), 0)
    tj = jax.lax.broadcasted_iota(jnp.int32, (T, T), 1)
    tril = (tj < ti).astype(jnp.float32)
    R = jnp.dot(tril, disp, preferred_element_type=jnp.float32)

    n_tiles = jnp.ceil(counts * (1.0 / BT))
    a8 = jax.lax.broadcasted_iota(jnp.int32, (NUM_EXPERTS, NUM_EXPERTS), 0)
    b8 = jax.lax.broadcasted_iota(jnp.int32, (NUM_EXPERTS, NUM_EXPERTS), 1)
    incl8 = (a8 <= b8).astype(jnp.float32)
    ends = jnp.dot(n_tiles, incl8, preferred_element_type=jnp.float32)
    row_start = (ends - n_tiles) * float(BT)

    rank1 = jnp.sum(oh1 * R, axis=1, keepdims=True)
    rank2 = jnp.sum(oh2 * R, axis=1, keepdims=True)
    base1 = jnp.sum(oh1 * row_start, axis=1, keepdims=True)
    base2 = jnp.sum(oh2 * row_start, axis=1, keepdims=True)
    pos0_ref[...] = (base1 + rank1).astype(jnp.int32)
    pos1_ref[...] = (base2 + rank2).astype(jnp.int32)

    nact_ref[...] = ends[:, NUM_EXPERTS - 1:].astype(jnp.int32)

    e8 = jax.lax.broadcasted_iota(jnp.int32, (1, NUM_EXPERTS), 1).astype(jnp.float32)
    last_e = jnp.max(jnp.where(n_tiles > 0, e8, -1.0), axis=1, keepdims=True)
    ii = jax.lax.broadcasted_iota(jnp.int32, (32, NUM_EXPERTS), 0).astype(jnp.float32)
    raw = jnp.sum((ends <= ii).astype(jnp.float32), axis=1, keepdims=True)
    meta_ref[...] = jnp.minimum(raw, last_e).astype(jnp.int32)


def _router(x, Wr):
    return pl.pallas_call(
        _router_body,
        out_shape=(
            jax.ShapeDtypeStruct((T, 1), jnp.int32),
            jax.ShapeDtypeStruct((T, 1), jnp.int32),
            jax.ShapeDtypeStruct((T, 1), jnp.float32),
            jax.ShapeDtypeStruct((T, 1), jnp.float32),
            jax.ShapeDtypeStruct((32, 1), jnp.int32),
            jax.ShapeDtypeStruct((1, 1), jnp.int32),
            jax.ShapeDtypeStruct((1, 1), jnp.float32),
        ),
    )(x, Wr)


@functools.cache
def _sc_mesh():
    return plsc.VectorSubcoreMesh(core_axis_name="c", subcore_axis_name="s")


def _dispatch_body(x_hbm, posr_hbm, xg_hbm, idx_v, rows_a, rows_b,
                   sin_a, sin_b, sout):
    wid = lax.axis_index("s") * NC + lax.axis_index("c")
    pltpu.sync_copy(posr_hbm.at[wid], idx_v)
    base = wid * TPW
    in_a = pltpu.async_copy(x_hbm.at[pl.ds(base, HALF)], rows_a, sin_a)
    in_b = pltpu.async_copy(x_hbm.at[pl.ds(base + HALF, HALF)], rows_b, sin_b)
    in_a.wait()
    s0 = pltpu.async_copy(rows_a, xg_hbm.at[idx_v.at[0]], sout)
    s1 = pltpu.async_copy(rows_a, xg_hbm.at[idx_v.at[1]], sout)
    in_b.wait()
    s2 = pltpu.async_copy(rows_b, xg_hbm.at[idx_v.at[2]], sout)
    s3 = pltpu.async_copy(rows_b, xg_hbm.at[idx_v.at[3]], sout)
    s0.wait()
    s1.wait()
    s2.wait()
    s3.wait()


def _dispatch(x, posr):
    return pl.kernel(
        _dispatch_body,
        out_type=jax.ShapeDtypeStruct((PADTOT, D_MODEL), jnp.float32),
        mesh=_sc_mesh(),
        scratch_types=[
            pltpu.VMEM((4, HALF), jnp.int32),
            pltpu.VMEM((HALF, D_MODEL), jnp.float32),
            pltpu.VMEM((HALF, D_MODEL), jnp.float32),
            pltpu.SemaphoreType.DMA,
            pltpu.SemaphoreType.DMA,
            pltpu.SemaphoreType.DMA,
        ],
    )(x, posr)


def _gmm_body(meta_ref, nact_ref, xg_ref, w1_ref, b1_ref, w2_ref, b2_ref, y_ref):
    i = pl.program_id(0)

    @pl.when(i < nact_ref[0])
    def _():
        h = jnp.maximum(
            jnp.dot(xg_ref[...], w1_ref[0], preferred_element_type=jnp.float32)
            + b1_ref[0],
            0.0,
        )
        y_ref[...] = jnp.dot(h, w2_ref[0], preferred_element_type=jnp.float32) + b2_ref[0]


def _gmm(xg, W1, b1, W2, b2, meta, nact):
    grid_spec = pltpu.PrefetchScalarGridSpec(
        num_scalar_prefetch=2,
        grid=(MAXT,),
        in_specs=[
            pl.BlockSpec((BT, D_MODEL),
                         lambda i, meta, nact: (jnp.minimum(i, nact[0] - 1), 0)),
            pl.BlockSpec((1, D_MODEL, D_FF), lambda i, meta, nact: (meta[i], 0, 0)),
            pl.BlockSpec((1, 1, D_FF), lambda i, meta, nact: (meta[i], 0, 0)),
            pl.BlockSpec((1, D_FF, D_MODEL), lambda i, meta, nact: (meta[i], 0, 0)),
            pl.BlockSpec((1, 1, D_MODEL), lambda i, meta, nact: (meta[i], 0, 0)),
        ],
        out_specs=pl.BlockSpec((BT, D_MODEL), lambda i, meta, nact: (i, 0)),
    )
    return pl.pallas_call(
        _gmm_body,
        grid_spec=grid_spec,
        out_shape=jax.ShapeDtypeStruct((PADTOT, D_MODEL), jnp.float32),
    )(meta, nact, xg, W1, b1.reshape(NUM_EXPERTS, 1, D_FF), W2,
      b2.reshape(NUM_EXPERTS, 1, D_MODEL))


def _gather_body(y_hbm, posr_hbm, ya_hbm, yb_hbm, idx_v, rows_v, sem):
    wid = lax.axis_index("s") * NC + lax.axis_index("c")
    pltpu.sync_copy(posr_hbm.at[wid], idx_v)
    base = wid * TPW
    for half in range(2):
        for k in range(2):
            pltpu.async_copy(y_hbm.at[idx_v.at[2 * half + k]], rows_v, sem).wait()
            dst = ya_hbm if k == 0 else yb_hbm
            pltpu.sync_copy(rows_v, dst.at[pl.ds(base + half * HALF, HALF)])


def _gather(y, posr):
    return pl.kernel(
        _gather_body,
        out_type=(
            jax.ShapeDtypeStruct((T, D_MODEL), jnp.float32),
            jax.ShapeDtypeStruct((T, D_MODEL), jnp.float32),
        ),
        mesh=_sc_mesh(),
        scratch_types=[
            pltpu.VMEM((4, HALF), jnp.int32),
            pltpu.VMEM((HALF, D_MODEL), jnp.float32),
            pltpu.SemaphoreType.DMA,
        ],
    )(y, posr)


def _combine_body(x_ref, ya_ref, yb_ref, w0_ref, w1_ref, out_ref):
    out_ref[...] = (x_ref[...] + w0_ref[...] * ya_ref[...]
                    + w1_ref[...] * yb_ref[...])


def _combine(x, ya, yb, w0, w1):
    nt = T // BT
    return pl.pallas_call(
        _combine_body,
        grid=(nt,),
        in_specs=[
            pl.BlockSpec((BT, D_MODEL), lambda t: (t, 0)),
            pl.BlockSpec((BT, D_MODEL), lambda t: (t, 0)),
            pl.BlockSpec((BT, D_MODEL), lambda t: (t, 0)),
            pl.BlockSpec((BT, 1), lambda t: (t, 0)),
            pl.BlockSpec((BT, 1), lambda t: (t, 0)),
        ],
        out_specs=pl.BlockSpec((BT, D_MODEL), lambda t: (t, 0)),
        out_shape=jax.ShapeDtypeStruct((T, D_MODEL), jnp.float32),
    )(x, ya, yb, w0, w1)


def kernel(input_batch, Wr, W1, b1, W2, b2):
    B, S, D = input_batch.shape
    x = input_batch.reshape(T, D)

    pos0, pos1, w0, w1, meta, nact, loss = _router(x, Wr)

    p = jnp.concatenate(
        [pos0.reshape(NW, 2, HALF, 1), pos1.reshape(NW, 2, HALF, 1)], axis=3)
    posr = p.transpose(0, 1, 3, 2).reshape(NW, 4, HALF)

    xg = _dispatch(x, posr)
    y = _gmm(xg, W1, b1, W2, b2, meta.reshape(32), nact.reshape(1))
    ya, yb = _gather(y, posr)
    out = _combine(x, ya, yb, w0, w1)

    return out.reshape(B, S, D), loss[0, 0]
